# baseline (speedup 1.0000x reference)
import functools

import jax
import jax.numpy as jnp
from jax import lax
from jax.experimental import pallas as pl
from jax.experimental.pallas import tpu as pltpu
from jax.experimental.pallas import tpu_sc as plsc

D_MODEL = 64
SCALE = 8.0
NUM_WORKERS = 32
CHUNK = 256
N_SEQ = 200
N_BATCH = 4096
ROW_UNROLL = 4


def _emb_body(idx_hbm, table_hbm, out_hbm, idx0, idx1, src0, src1, tb0, tb1,
              ob0, ob1, isem0, isem1, gsem0, gsem1, osem0, osem1, *, n_chunks):
    wid = lax.axis_index("s") * 2 + lax.axis_index("c")
    c0 = wid * n_chunks
    idxb = (idx0, idx1)
    srcb = (src0, src1)
    tbb = (tb0, tb1)
    obb = (ob0, ob1)
    isem = (isem0, isem1)
    gsem = (gsem0, gsem1)
    osem = (osem0, osem1)
    chunks_per_s = N_BATCH // CHUNK
    bt_per_chunk = CHUNK // 128

    def start_idx(j, b):
        c = c0 + j
        pltpu.async_copy(
            idx_hbm.at[c // chunks_per_s,
                       pl.ds((c % chunks_per_s) * CHUNK, CHUNK)],
            idxb[b], isem[b],
        )

    def start_gather(b):
        pltpu.async_copy(table_hbm.at[idxb[b]], srcb[b], gsem[b])

    pltpu.sync_copy(
        idx_hbm.at[c0 // chunks_per_s, pl.ds((c0 % chunks_per_s) * CHUNK, CHUNK)],
        idx0,
    )
    start_gather(0)
    start_idx(1, 1)

    def chunk_body(j, carry):
        b = lax.rem(j, 2)
        c = c0 + j
        s = c // chunks_per_s
        bt0 = (c % chunks_per_s) * bt_per_chunk

        def on_buf(bb):
            pltpu.make_async_copy(
                table_hbm.at[idxb[bb]], srcb[bb], gsem[bb]
            ).wait()

            @pl.when(j + 1 < n_chunks)
            def _():
                pltpu.make_async_copy(
                    idx_hbm.at[0, pl.ds(0, CHUNK)], idxb[1 - bb], isem[1 - bb]
                ).wait()
                pltpu.async_copy(
                    table_hbm.at[idxb[1 - bb]], srcb[1 - bb], gsem[1 - bb]
                )

            @pl.when(j + 2 < n_chunks)
            def _():
                start_idx(j + 2, bb)

            @pl.when(j >= 2)
            def _():
                for bts in range(bt_per_chunk):
                    pltpu.make_async_copy(
                        obb[bb].at[bts], out_hbm.at[0, :, 0], osem[bb]
                    ).wait()

            lanes = lax.iota(jnp.int32, 16)
            dtvecs = [(lanes // 8) + 2 * jj for jj in range(D_MODEL // 16)]
            drvec = lanes % 8

            def row_body(r4, gc):
                r0 = r4 * ROW_UNROLL
                for dr_ in range(ROW_UNROLL):
                    r = r0 + dr_
                    btsv = jnp.full((16,), r // 128, jnp.int32)
                    bcv = jnp.full((16,), r % 128, jnp.int32)
                    for jj in range(D_MODEL // 16):
                        v = srcb[bb][r, pl.ds(jj * 16, 16)] * SCALE
                        plsc.store_scatter(
                            tbb[bb], [btsv, dtvecs[jj], drvec, bcv], v
                        )
                return gc

            lax.fori_loop(0, CHUNK // ROW_UNROLL, row_body, 0)

            def repack_body(dt, gc):
                for bts in range(bt_per_chunk):
                    for dr_ in range(8):
                        for k in range(8):
                            sl = pl.ds(k * 16, 16)
                            obb[bb][bts, dt, dr_, sl] = tbb[bb][bts, dt, dr_, sl]
                return gc

            lax.fori_loop(0, 8, repack_body, 0)
            for bts in range(bt_per_chunk):
                pltpu.async_copy(
                    obb[bb].at[bts], out_hbm.at[s, :, bt0 + bts], osem[bb]
                )

        @pl.when(b == 0)
        def _():
            on_buf(0)

        @pl.when(b == 1)
        def _():
            on_buf(1)

        return carry

    lax.fori_loop(0, n_chunks, chunk_body, 0)
    for b in range(2):
        for bts in range(bt_per_chunk):
            pltpu.make_async_copy(
                obb[b].at[bts], out_hbm.at[0, :, 0], osem[b]
            ).wait()


def kernel(x, emb_weight):
    k_total = x.size
    n_chunks = k_total // (NUM_WORKERS * CHUNK)
    xt = x.T.astype(jnp.int32)

    mesh = plsc.VectorSubcoreMesh(core_axis_name="c", subcore_axis_name="s")

    emb = functools.partial(
        pl.kernel,
        mesh=mesh,
        out_type=jax.ShapeDtypeStruct(
            (N_SEQ, D_MODEL // 8, N_BATCH // 128, 8, 128), jnp.float32
        ),
        scratch_types=[
            pltpu.VMEM((CHUNK,), jnp.int32),
            pltpu.VMEM((CHUNK,), jnp.int32),
            pltpu.VMEM((CHUNK, D_MODEL), jnp.float32),
            pltpu.VMEM((CHUNK, D_MODEL), jnp.float32),
            pltpu.VMEM((CHUNK // 128, D_MODEL // 8, 8, 129), jnp.float32),
            pltpu.VMEM((CHUNK // 128, D_MODEL // 8, 8, 129), jnp.float32),
            pltpu.VMEM((CHUNK // 128, D_MODEL // 8, 8, 128), jnp.float32),
            pltpu.VMEM((CHUNK // 128, D_MODEL // 8, 8, 128), jnp.float32),
            pltpu.SemaphoreType.DMA,
            pltpu.SemaphoreType.DMA,
            pltpu.SemaphoreType.DMA,
            pltpu.SemaphoreType.DMA,
            pltpu.SemaphoreType.DMA,
            pltpu.SemaphoreType.DMA,
        ],
        compiler_params=pltpu.CompilerParams(
            use_tc_tiling_on_sc=False, needs_layout_passes=False
        ),
    )(functools.partial(_emb_body, n_chunks=n_chunks))

    out5 = emb(xt, emb_weight)
    return out5.transpose(2, 4, 0, 1, 3).reshape(N_BATCH, N_SEQ, D_MODEL)


# speedup vs baseline: 1.3680x; 1.3680x over previous
import functools

import jax
import jax.numpy as jnp
from jax import lax
from jax.experimental import pallas as pl
from jax.experimental.pallas import tpu as pltpu
from jax.experimental.pallas import tpu_sc as plsc

D_MODEL = 64
SCALE = 8.0
NUM_WORKERS = 32
CHUNK = 256
N_SEQ = 200
N_BATCH = 4096
ROW_UNROLL = 4


def _emb_body(idx_hbm, table_hbm, out_hbm, idx0, idx1, src0, src1,
              tb0, tb1, isem0, isem1, gsem0, gsem1, osem0, osem1, *, n_chunks):
    wid = lax.axis_index("s") * 2 + lax.axis_index("c")
    c0 = wid * n_chunks
    idxb = (idx0, idx1)
    srcb = (src0, src1)
    tbb = (tb0, tb1)
    isem = (isem0, isem1)
    gsem = (gsem0, gsem1)
    osem = (osem0, osem1)
    chunks_per_s = N_BATCH // CHUNK
    bt_per_chunk = CHUNK // 128

    def start_idx(j, b):
        c = c0 + j
        pltpu.async_copy(
            idx_hbm.at[c // chunks_per_s,
                       pl.ds((c % chunks_per_s) * CHUNK, CHUNK)],
            idxb[b], isem[b],
        )

    def start_gather(b):
        pltpu.async_copy(table_hbm.at[idxb[b]], srcb[b], gsem[b])

    pltpu.sync_copy(
        idx_hbm.at[c0 // chunks_per_s, pl.ds((c0 % chunks_per_s) * CHUNK, CHUNK)],
        idx0,
    )
    start_gather(0)
    start_idx(1, 1)

    def chunk_body(j, carry):
        b = lax.rem(j, 2)
        c = c0 + j
        s = c // chunks_per_s
        bt0 = (c % chunks_per_s) * bt_per_chunk

        def on_buf(bb):
            pltpu.make_async_copy(
                table_hbm.at[idxb[bb]], srcb[bb], gsem[bb]
            ).wait()

            @pl.when(j + 1 < n_chunks)
            def _():
                pltpu.make_async_copy(
                    idx_hbm.at[0, pl.ds(0, CHUNK)], idxb[1 - bb], isem[1 - bb]
                ).wait()
                pltpu.async_copy(
                    table_hbm.at[idxb[1 - bb]], srcb[1 - bb], gsem[1 - bb]
                )

            @pl.when(j + 2 < n_chunks)
            def _():
                start_idx(j + 2, bb)

            @pl.when(j >= 2)
            def _():
                for bts in range(bt_per_chunk):
                    pltpu.make_async_copy(
                        tbb[bb].at[bts, :, :, pl.ds(0, 128)],
                        out_hbm.at[0, :, 0],
                        osem[bb],
                    ).wait()

            lanes = lax.iota(jnp.int32, 16)
            dtvecs = [(lanes // 8) + 2 * jj for jj in range(D_MODEL // 16)]
            drvec = lanes % 8

            def row_body(g, gc):
                r0 = g * 16
                btsv = jnp.full((16,), g // 8, jnp.int32)
                bc0 = (g % 8) * 16
                for l in range(16):
                    r = r0 + l
                    bcv = jnp.full((16,), bc0 + l, jnp.int32)
                    for jj in range(D_MODEL // 16):
                        v = srcb[bb][r, pl.ds(jj * 16, 16)] * SCALE
                        plsc.store_scatter(
                            tbb[bb], [btsv, dtvecs[jj], drvec, bcv], v
                        )
                return gc

            lax.fori_loop(0, CHUNK // 16, row_body, 0)
            for bts in range(bt_per_chunk):
                pltpu.async_copy(
                    tbb[bb].at[bts, :, :, pl.ds(0, 128)],
                    out_hbm.at[s, :, bt0 + bts],
                    osem[bb],
                )

        @pl.when(b == 0)
        def _():
            on_buf(0)

        @pl.when(b == 1)
        def _():
            on_buf(1)

        return carry

    lax.fori_loop(0, n_chunks, chunk_body, 0)
    for b in range(2):
        for bts in range(bt_per_chunk):
            pltpu.make_async_copy(
                tbb[b].at[bts, :, :, pl.ds(0, 128)], out_hbm.at[0, :, 0], osem[b]
            ).wait()


def kernel(x, emb_weight):
    k_total = x.size
    n_chunks = k_total // (NUM_WORKERS * CHUNK)
    xt = x.T.astype(jnp.int32)
    wt = jnp.pad(emb_weight, ((0, 0), (0, D_MODEL)))

    mesh = plsc.VectorSubcoreMesh(core_axis_name="c", subcore_axis_name="s")

    emb = functools.partial(
        pl.kernel,
        mesh=mesh,
        out_type=jax.ShapeDtypeStruct(
            (N_SEQ, D_MODEL // 8, N_BATCH // 128, 8, 128), jnp.float32
        ),
        scratch_types=[
            pltpu.VMEM((CHUNK,), jnp.int32),
            pltpu.VMEM((CHUNK,), jnp.int32),
            pltpu.VMEM((CHUNK, 2 * D_MODEL), jnp.float32),
            pltpu.VMEM((CHUNK, 2 * D_MODEL), jnp.float32),
            pltpu.VMEM((CHUNK // 128, D_MODEL // 8, 8, 129), jnp.float32),
            pltpu.VMEM((CHUNK // 128, D_MODEL // 8, 8, 129), jnp.float32),
            pltpu.SemaphoreType.DMA,
            pltpu.SemaphoreType.DMA,
            pltpu.SemaphoreType.DMA,
            pltpu.SemaphoreType.DMA,
            pltpu.SemaphoreType.DMA,
            pltpu.SemaphoreType.DMA,
        ],
        compiler_params=pltpu.CompilerParams(
            use_tc_tiling_on_sc=False, needs_layout_passes=False
        ),
    )(functools.partial(_emb_body, n_chunks=n_chunks))

    out5 = emb(xt, wt)
    return out5.transpose(2, 4, 0, 1, 3).reshape(N_BATCH, N_SEQ, D_MODEL)
